# baseline (speedup 1.0000x reference)
import functools

import jax
import jax.numpy as jnp
from jax import lax
from jax.experimental import pallas as pl
from jax.experimental.pallas import tpu as pltpu
from jax.experimental.pallas import tpu_sc as plsc

_NUM_HEADS = 16
_P_PAD = 640
_HALF = _P_PAD // 2
_LANES = 16
_CHUNKS = _HALF // _LANES


def _sc_bias_gather(table_flat, idx_pad):
    mesh = plsc.VectorSubcoreMesh(core_axis_name="c", subcore_axis_name="s")

    @functools.partial(
        pl.kernel,
        mesh=mesh,
        out_type=jax.ShapeDtypeStruct((_NUM_HEADS * _P_PAD,), jnp.float32),
        scratch_types=[
            pltpu.VMEM((table_flat.shape[0],), jnp.float32),
            pltpu.VMEM((_HALF,), jnp.int32),
            pltpu.VMEM((_HALF,), jnp.float32),
        ],
        compiler_params=pltpu.CompilerParams(needs_layout_passes=False),
    )
    def body(table_hbm, idx_hbm, out_hbm, table_v, idx_v, out_v):
        wid = lax.axis_index("s") * 2 + lax.axis_index("c")
        head = wid // 2
        half = wid % 2
        pltpu.sync_copy(table_hbm, table_v)
        pltpu.sync_copy(idx_hbm.at[pl.ds(half * _HALF, _HALF)], idx_v)
        for j in range(_CHUNKS):
            iv = idx_v[pl.ds(j * _LANES, _LANES)]
            g = iv * _NUM_HEADS + head
            out_v[pl.ds(j * _LANES, _LANES)] = plsc.load_gather(table_v, [g])
        pltpu.sync_copy(
            out_v, out_hbm.at[pl.ds(head * _P_PAD + half * _HALF, _HALF)]
        )

    return body(table_flat, idx_pad)


def kernel(bias_table, rel_idx):
    grid_sq = rel_idx.shape[0] * rel_idx.shape[1]
    table_flat = bias_table.reshape(-1)
    idx_pad = jnp.pad(
        rel_idx.reshape(-1).astype(jnp.int32), (0, _P_PAD - grid_sq)
    )
    out_flat = _sc_bias_gather(table_flat, idx_pad)
    out = out_flat.reshape(_NUM_HEADS, _P_PAD)[:, :grid_sq]
    return out.reshape(_NUM_HEADS, rel_idx.shape[0], rel_idx.shape[1])
